# Initial kernel scaffold; baseline (speedup 1.0000x reference)
import functools

import jax
import jax.numpy as jnp
from jax import lax
from jax.experimental import pallas as pl
from jax.experimental.pallas import tpu as pltpu
from jax.experimental.pallas import tpu_sc as plsc

L = 16
D = 8
NC = 2
NS = 16
NW = NC * NS


@functools.partial(jax.jit, static_argnums=0)
def _noop(n):
    return n


def _build_call(n):
    groups = n // L
    gbase = groups // NW
    extra = groups - gbase * NW
    wcut = NW - extra
    gmax = gbase + (1 if extra else 0)
    rows_w = gbase * L
    rows_max = gmax * L

    g1 = groups // NS
    rem1 = groups - g1 * NS
    ch1 = g1 * L
    tail_off = ch1 * NS

    mesh = plsc.VectorSubcoreMesh(core_axis_name="c", subcore_axis_name="s")

    @functools.partial(
        pl.kernel,
        out_type=jax.ShapeDtypeStruct((n * D,), jnp.float32),
        mesh=mesh,
        scratch_types=[
            pltpu.VMEM((ch1,), jnp.int32),
            pltpu.VMEM((L,), jnp.int32),
            pltpu.VMEM((rows_max,), jnp.int32),
            pltpu.VMEM((rows_max,), jnp.float32),
            pltpu.VMEM((rows_max * D,), jnp.float32),
            pltpu.VMEM((L,), jnp.int32),
            pltpu.VMEM((NS * L,), jnp.int32),
            pltpu.VMEM((L,), jnp.int32),
            pltpu.VMEM_SHARED((NS * L,), jnp.int32),
        ],
    )
    def call(ls_hbm, sv_hbm, lim_hbm, out_hbm,
             vals1_v, tail_v, vals2_v, src_v, out_v,
             minvec_v, allmins_v, lim_v, mins_sh):
        c = lax.axis_index("c")
        s = lax.axis_index("s")
        w = s * NC + c

        pltpu.sync_copy(ls_hbm.at[pl.ds(s * ch1, ch1)], vals1_v)
        if rem1:
            pltpu.sync_copy(
                ls_hbm.at[pl.ds(tail_off + lax.rem(s, rem1) * L, L)], tail_v)
            m0 = tail_v[...]
        else:
            m0 = jnp.full((L,), jnp.iinfo(jnp.int32).max, jnp.int32)

        def mstep(i, m):
            return jnp.minimum(m, vals1_v[pl.ds(i * L, L)])
        m = lax.fori_loop(0, g1, mstep, m0)
        minvec_v[...] = m
        pltpu.sync_copy(minvec_v, mins_sh.at[pl.ds(s * L, L)])
        plsc.subcore_barrier()
        pltpu.sync_copy(mins_sh, allmins_v)

        def mstep2(i, m):
            return jnp.minimum(m, allmins_v[pl.ds(i * L, L)])
        mall = lax.fori_loop(0, NS, mstep2,
                             jnp.full((L,), jnp.iinfo(jnp.int32).max,
                                      jnp.int32))
        gmin = jnp.min(mall)

        base = rows_w * w + L * jnp.maximum(w - wcut, 0)
        ng = gbase + jnp.where(w >= wcut, 1, 0) if extra else gbase
        pltpu.sync_copy(ls_hbm.at[pl.ds(base, rows_max)], vals2_v)
        pltpu.sync_copy(sv_hbm.at[pl.ds(base, rows_max)], src_v)
        pltpu.sync_copy(lim_hbm, lim_v)
        lim = lim_v[...]
        lane8 = lax.iota(jnp.int32, L) * D
        zeros = jnp.zeros((L,), jnp.float32)

        def wstep(i, carry):
            b = i * (L * D)
            for j in range(D):
                out_v[pl.ds(b + j * L, L)] = zeros
            v = vals2_v[pl.ds(i * L, L)] - gmin
            v = jnp.minimum(v, lim)
            v = jnp.maximum(v, 0)
            plsc.store_scatter(out_v, [b + lane8 + v],
                               src_v[pl.ds(i * L, L)])
            return carry
        lax.fori_loop(0, ng, wstep, 0)

        wout = rows_w * D
        pltpu.sync_copy(out_v.at[pl.ds(0, wout)],
                        out_hbm.at[pl.ds(base * D, wout)])
        if extra:
            @pl.when(w >= wcut)
            def _():
                pltpu.sync_copy(out_v.at[pl.ds(wout, L * D)],
                                out_hbm.at[pl.ds(base * D + wout, L * D)])

    return call


def kernel(list_scalars, src_vals, one_hot_dim):
    n = list_scalars.shape[0]
    lim = jnp.full((L,), jnp.minimum(one_hot_dim - 1, D - 1), jnp.int32)
    out_flat = _build_call(n)(list_scalars, src_vals, lim)
    return out_flat.reshape(n, D)


# speedup vs baseline: 5.2926x; 5.2926x over previous
import functools

import jax
import jax.numpy as jnp
from jax import lax
from jax.experimental import pallas as pl
from jax.experimental.pallas import tpu as pltpu
from jax.experimental.pallas import tpu_sc as plsc

L = 16
D = 8
NC = 2
NS = 16
NW = NC * NS


def _build_call(n):
    groups = n // L
    gbase = groups // NW
    extra = groups - gbase * NW
    wcut = NW - extra
    gmax = gbase + (1 if extra else 0)
    rows_w = gbase * L
    rows_max = gmax * L

    g1 = groups // NS
    rem1 = groups - g1 * NS
    ch1 = g1 * L
    tail_off = ch1 * NS

    mesh = plsc.VectorSubcoreMesh(core_axis_name="c", subcore_axis_name="s")

    @functools.partial(
        pl.kernel,
        out_type=jax.ShapeDtypeStruct((n * D,), jnp.float32),
        mesh=mesh,
        scratch_types=[
            pltpu.VMEM((ch1,), jnp.int32),
            pltpu.VMEM((L,), jnp.int32),
            pltpu.VMEM((rows_max,), jnp.int32),
            pltpu.VMEM((rows_max,), jnp.float32),
            pltpu.VMEM((rows_max * D,), jnp.float32),
            pltpu.VMEM((L,), jnp.int32),
            pltpu.VMEM((NS * L,), jnp.int32),
            pltpu.VMEM((L,), jnp.int32),
            pltpu.VMEM_SHARED((NS * L,), jnp.int32),
        ],
        compiler_params=pltpu.CompilerParams(needs_layout_passes=False),
    )
    def call(ls_hbm, sv_hbm, lim_hbm, out_hbm,
             vals1_v, tail_v, vals2_v, src_v, out_v,
             minvec_v, allmins_v, lim_v, mins_sh):
        c = lax.axis_index("c")
        s = lax.axis_index("s")
        w = s * NC + c

        pltpu.sync_copy(ls_hbm.at[pl.ds(s * ch1, ch1)], vals1_v)
        if rem1:
            pltpu.sync_copy(
                ls_hbm.at[pl.ds(tail_off + lax.rem(s, rem1) * L, L)], tail_v)
            m0 = tail_v[...]
        else:
            m0 = jnp.full((L,), jnp.iinfo(jnp.int32).max, jnp.int32)

        def mstep(i, m):
            return jnp.minimum(m, vals1_v[pl.ds(i * L, L)])
        m = lax.fori_loop(0, g1, mstep, m0)
        minvec_v[...] = m
        pltpu.sync_copy(minvec_v, mins_sh.at[pl.ds(s * L, L)])
        plsc.subcore_barrier()
        pltpu.sync_copy(mins_sh, allmins_v)

        def mstep2(i, m):
            return jnp.minimum(m, allmins_v[pl.ds(i * L, L)])
        mall = lax.fori_loop(0, NS, mstep2,
                             jnp.full((L,), jnp.iinfo(jnp.int32).max,
                                      jnp.int32))
        gmin = mall[0]
        for j in range(1, L):
            gmin = jnp.minimum(gmin, mall[j])

        base = rows_w * w + L * jnp.maximum(w - wcut, 0)
        ng = gbase + jnp.where(w >= wcut, 1, 0) if extra else gbase
        pltpu.sync_copy(ls_hbm.at[pl.ds(base, rows_max)], vals2_v)
        pltpu.sync_copy(sv_hbm.at[pl.ds(base, rows_max)], src_v)
        pltpu.sync_copy(lim_hbm, lim_v)
        lim = lim_v[...]
        lane8 = lax.iota(jnp.int32, L) * D
        zeros = jnp.zeros((L,), jnp.float32)

        def wstep(i, carry):
            b = i * (L * D)
            for j in range(D):
                out_v[pl.ds(b + j * L, L)] = zeros
            v = vals2_v[pl.ds(i * L, L)] - gmin
            v = jnp.minimum(v, lim)
            v = jnp.maximum(v, 0)
            plsc.store_scatter(out_v, [b + lane8 + v],
                               src_v[pl.ds(i * L, L)])
            return carry
        lax.fori_loop(0, ng, wstep, 0)

        wout = rows_w * D
        pltpu.sync_copy(out_v.at[pl.ds(0, wout)],
                        out_hbm.at[pl.ds(base * D, wout)])
        if extra:
            @pl.when(w >= wcut)
            def _():
                pltpu.sync_copy(out_v.at[pl.ds(wout, L * D)],
                                out_hbm.at[pl.ds(base * D + wout, L * D)])

    return call


def kernel(list_scalars, src_vals, one_hot_dim):
    n = list_scalars.shape[0]
    lim = jnp.full((L,), jnp.minimum(one_hot_dim - 1, D - 1), jnp.int32)
    out_flat = _build_call(n)(list_scalars, src_vals, lim)
    return out_flat.reshape(n, D)


# speedup vs baseline: 6.4404x; 1.2169x over previous
import functools

import jax
import jax.numpy as jnp
from jax import lax
from jax.experimental import pallas as pl
from jax.experimental.pallas import tpu as pltpu
from jax.experimental.pallas import tpu_sc as plsc

L = 16
D = 8
NC = 2
NS = 16
NW = NC * NS


def _build_call(n):
    groups = n // L
    gbase = groups // NW
    extra = groups - gbase * NW
    wcut = NW - extra
    gmax = gbase + (1 if extra else 0)
    rows_w = gbase * L
    rows_max = gmax * L

    g1 = groups // NS
    rem1 = groups - g1 * NS
    ch1 = g1 * L
    tail_off = ch1 * NS

    chg = 13
    assert gbase % chg == 0
    nch = gbase // chg
    chr_ = chg * L

    mesh = plsc.VectorSubcoreMesh(core_axis_name="c", subcore_axis_name="s")

    @functools.partial(
        pl.kernel,
        out_type=jax.ShapeDtypeStruct((n, D), jnp.float32),
        mesh=mesh,
        scratch_types=[
            pltpu.VMEM((ch1,), jnp.int32),
            pltpu.VMEM((L,), jnp.int32),
            pltpu.VMEM((rows_max,), jnp.int32),
            pltpu.VMEM((rows_max,), jnp.float32),
            pltpu.VMEM((chr_, D), jnp.float32),
            pltpu.VMEM((L,), jnp.int32),
            pltpu.VMEM((NS * L,), jnp.int32),
            pltpu.VMEM((L,), jnp.int32),
            pltpu.VMEM_SHARED((NS * L,), jnp.int32),
        ],
        compiler_params=pltpu.CompilerParams(needs_layout_passes=False),
    )
    def call(ls_hbm, sv_hbm, lim_hbm, out_hbm,
             vals1_v, tail_v, vals2_v, src_v, out_v,
             minvec_v, allmins_v, lim_v, mins_sh):
        c = lax.axis_index("c")
        s = lax.axis_index("s")
        w = s * NC + c

        pltpu.sync_copy(ls_hbm.at[pl.ds(s * ch1, ch1)], vals1_v)
        if rem1:
            pltpu.sync_copy(
                ls_hbm.at[pl.ds(tail_off + lax.rem(s, rem1) * L, L)], tail_v)
            m0 = tail_v[...]
        else:
            m0 = jnp.full((L,), jnp.iinfo(jnp.int32).max, jnp.int32)

        def mstep(i, m):
            return jnp.minimum(m, vals1_v[pl.ds(i * L, L)])
        m = lax.fori_loop(0, g1, mstep, m0)
        minvec_v[...] = m
        pltpu.sync_copy(minvec_v, mins_sh.at[pl.ds(s * L, L)])
        plsc.subcore_barrier()
        pltpu.sync_copy(mins_sh, allmins_v)

        def mstep2(i, m):
            return jnp.minimum(m, allmins_v[pl.ds(i * L, L)])
        mall = lax.fori_loop(0, NS, mstep2,
                             jnp.full((L,), jnp.iinfo(jnp.int32).max,
                                      jnp.int32))
        gmin = mall[0]
        for j in range(1, L):
            gmin = jnp.minimum(gmin, mall[j])

        base = rows_w * w + L * jnp.maximum(w - wcut, 0)
        ng = gbase + jnp.where(w >= wcut, 1, 0) if extra else gbase
        pltpu.sync_copy(ls_hbm.at[pl.ds(base, rows_max)], vals2_v)
        pltpu.sync_copy(sv_hbm.at[pl.ds(base, rows_max)], src_v)
        pltpu.sync_copy(lim_hbm, lim_v)
        lim = lim_v[...]
        lane = lax.iota(jnp.int32, L)
        colpat = jnp.bitwise_and(lane, D - 1)
        rsub = lax.shift_right_logical(lane, 3)
        zeros = jnp.zeros((L,), jnp.float32)

        def fill_group(rbase, gi):
            for j in range(D):
                plsc.store_scatter(out_v, [rbase + (j * 2) + rsub, colpat],
                                   zeros)
            v = vals2_v[pl.ds(gi * L, L)] - gmin
            v = jnp.minimum(v, lim)
            v = jnp.maximum(v, 0)
            plsc.store_scatter(out_v, [rbase + lane, v],
                               src_v[pl.ds(gi * L, L)])

        def cstep(k, carry):
            def gstep(g, carry2):
                fill_group(g * L, k * chg + g)
                return carry2
            lax.fori_loop(0, chg, gstep, 0)
            pltpu.sync_copy(out_v,
                            out_hbm.at[pl.ds(base + k * chr_, chr_)])
            return carry
        lax.fori_loop(0, nch, cstep, 0)

        if extra:
            @pl.when(w >= wcut)
            def _():
                fill_group(0, gbase)
                pltpu.sync_copy(out_v.at[pl.ds(0, L)],
                                out_hbm.at[pl.ds(base + rows_w, L)])

    return call


def kernel(list_scalars, src_vals, one_hot_dim):
    n = list_scalars.shape[0]
    lim = jnp.full((L,), jnp.minimum(one_hot_dim - 1, D - 1), jnp.int32)
    return _build_call(n)(list_scalars, src_vals, lim)
